# baseline (speedup 1.0000x reference)
import functools

import jax
import jax.numpy as jnp
from jax import lax
from jax.experimental import pallas as pl
from jax.experimental.pallas import tpu as pltpu
from jax.experimental.pallas import tpu_sc as plsc

ROWS = 20
D_OUT = 256
RG = 5
CB = 64
NCOL_G = D_OUT // CB
NC = 1
LANES = 16
NVEC = CB // LANES

_mesh = plsc.VectorSubcoreMesh(core_axis_name="c", subcore_axis_name="s",
                               num_cores=NC)

_GATHER_DNUMS = lax.GatherDimensionNumbers(
    offset_dims=(), collapsed_slice_dims=(0,), start_index_map=(0,))


def _lane_broadcast(vec, lane_idx):
    return lax.gather(vec, lane_idx[:, None], _GATHER_DNUMS, (1,),
                      mode=lax.GatherScatterMode.PROMISE_IN_BOUNDS)


@functools.partial(
    pl.kernel,
    mesh=_mesh,
    compiler_params=pltpu.CompilerParams(use_tc_tiling_on_sc=False,
                                         needs_layout_passes=False),
    out_type=jax.ShapeDtypeStruct((ROWS, D_OUT), jnp.float32),
    scratch_types=[
        pltpu.VMEM((ROWS,), jnp.int32),
        pltpu.VMEM((ROWS,), jnp.int32),
        pltpu.VMEM((8, 42), jnp.float32),
        pltpu.VMEM((8, 42), jnp.float32),
        pltpu.VMEM((3, 44), jnp.float32),
        pltpu.VMEM((12, 42), jnp.float32),
        pltpu.VMEM((12, 42), jnp.float32),
        pltpu.VMEM((2, 44), jnp.float32),
        pltpu.VMEM((128, CB), jnp.float32),
        pltpu.VMEM((CB,), jnp.float32),
        pltpu.VMEM((RG, CB), jnp.float32),
        pltpu.SemaphoreType.DMA,
        pltpu.SemaphoreType.DMA,
        pltpu.SemaphoreType.DMA,
    ],
)
def _sc_embed_project(pid_hbm, oid_hbm, cslot_hbm, cpiece_hbm, corient_hbm,
                      eslot_hbm, epiece_hbm, eorient_hbm, w_hbm, b_hbm,
                      out_hbm,
                      pid_v, oid_v, cslot_v, cpiece_v, corient_v,
                      eslot_v, epiece_v, eorient_v, w_v, b_v, acc_v,
                      sem_ids, sem_tab, sem_w):
    wid = lax.axis_index("s") * NC + lax.axis_index("c")
    rg = wid // NCOL_G
    cg = wid % NCOL_G
    r0 = rg * RG
    c0 = cg * CB

    cp_ids = [pltpu.async_copy(pid_hbm, pid_v, sem_ids),
              pltpu.async_copy(oid_hbm, oid_v, sem_ids)]
    cp_tab = [pltpu.async_copy(cslot_hbm, cslot_v, sem_tab),
              pltpu.async_copy(cpiece_hbm, cpiece_v, sem_tab),
              pltpu.async_copy(corient_hbm, corient_v, sem_tab),
              pltpu.async_copy(eslot_hbm, eslot_v, sem_tab),
              pltpu.async_copy(epiece_hbm, epiece_v, sem_tab),
              pltpu.async_copy(eorient_hbm, eorient_v, sem_tab)]
    cp_w = [pltpu.async_copy(w_hbm.at[:, pl.ds(c0, CB)], w_v, sem_w),
            pltpu.async_copy(b_hbm.at[pl.ds(c0, CB)], b_v, sem_w)]

    lane_iota = lax.iota(jnp.int32, 16)

    for cp in cp_ids:
        cp.wait()
    rows_slot, rows_piece, rows_orient = [], [], []
    for r in range(RG):
        row = r0 + r
        row_splat = jnp.full((16,), row, jnp.int32)
        is_edge = row_splat >= 8
        pid_splat = plsc.load_gather(pid_v, [row_splat])
        oid_splat = plsc.load_gather(oid_v, [row_splat])
        rows_slot.append((is_edge,
                          jnp.minimum(row_splat, 7),
                          jnp.maximum(row_splat - 8, 0)))
        rows_piece.append((is_edge,
                           jnp.minimum(pid_splat, 7),
                           jnp.clip(pid_splat - 8, 0, 11)))
        rows_orient.append((is_edge, oid_splat, oid_splat))

    for cp in cp_w:
        cp.wait()
    acc = [[b_v[pl.ds(j * LANES, LANES)] for j in range(NVEC)]
           for _ in range(RG)]

    for cp in cp_tab:
        cp.wait()
    for ctab, etab, tab_rows, base_k, width in (
            (cslot_v, eslot_v, rows_slot, 0, 42),
            (cpiece_v, epiece_v, rows_piece, 42, 42),
            (corient_v, eorient_v, rows_orient, 84, 44)):
        for kb in range((width + LANES - 1) // LANES):
            lo = kb * LANES
            col_idx = jnp.minimum(lane_iota + lo, width - 1)
            evs = []
            for r in range(RG):
                is_edge, crow, erow = tab_rows[r]
                ec = plsc.load_gather(ctab, [crow, col_idx])
                ee = plsc.load_gather(etab, [erow, col_idx])
                evs.append(jnp.where(is_edge, ee, ec))
            for kl in range(lo, min(lo + LANES, width)):
                k = base_k + kl
                w_vecs = [w_v[k, pl.ds(j * LANES, LANES)]
                          for j in range(NVEC)]
                lane = jnp.full((16,), kl - lo, jnp.int32)
                for r in range(RG):
                    e_b = _lane_broadcast(evs[r], lane)
                    for j in range(NVEC):
                        acc[r][j] = acc[r][j] + e_b * w_vecs[j]

    for r in range(RG):
        for j in range(NVEC):
            acc_v[r, pl.ds(j * LANES, LANES)] = acc[r][j]
    pltpu.sync_copy(acc_v, out_hbm.at[pl.ds(r0, RG), pl.ds(c0, CB)])


def kernel(piece_ids, orientations, corner_slot_w, corner_piece_w,
           corner_orient_w, edge_slot_w, edge_piece_w, edge_orient_w,
           proj_w, proj_b):
    out = _sc_embed_project(piece_ids.reshape(ROWS), orientations.reshape(ROWS),
                            corner_slot_w, corner_piece_w, corner_orient_w,
                            edge_slot_w, edge_piece_w, edge_orient_w,
                            proj_w, proj_b)
    return out.reshape(1, ROWS, D_OUT)


# speedup vs baseline: 5.0000x; 5.0000x over previous
import jax
import jax.numpy as jnp
from jax.experimental import pallas as pl


def _fused_kernel(pid_ref, orient_ref, cslot_ref, cpiece_ref, corient_ref,
                  eslot_ref, epiece_ref, eorient_ref, proj_w_ref, proj_b_ref,
                  out_ref):
    pid = pid_ref[...]
    oid = orient_ref[...]

    slot_all = jnp.concatenate([cslot_ref[...], eslot_ref[...]], axis=0)
    piece_all = jnp.concatenate([cpiece_ref[...], epiece_ref[...]], axis=0)
    orient_all = jnp.concatenate([corient_ref[...], eorient_ref[...]], axis=0)

    row = jax.lax.broadcasted_iota(jnp.int32, (20, 1), 0)
    oid_adj = oid + jnp.where(row >= 8, 3, 0)

    k20 = jax.lax.broadcasted_iota(jnp.int32, (20, 20), 1)
    onehot_p = (pid == k20).astype(jnp.float32)
    emb_piece = jnp.dot(onehot_p, piece_all,
                        preferred_element_type=jnp.float32)

    k5 = jax.lax.broadcasted_iota(jnp.int32, (20, 5), 1)
    onehot_o = (oid_adj == k5).astype(jnp.float32)
    emb_orient = jnp.dot(onehot_o, orient_all,
                         preferred_element_type=jnp.float32)

    embedded = jnp.concatenate(
        [slot_all, emb_piece, emb_orient], axis=1)

    out_ref[...] = (jnp.dot(embedded, proj_w_ref[...],
                            preferred_element_type=jnp.float32)
                    + proj_b_ref[...])


def kernel(piece_ids, orientations, corner_slot_w, corner_piece_w,
           corner_orient_w, edge_slot_w, edge_piece_w, edge_orient_w,
           proj_w, proj_b):
    out = pl.pallas_call(
        _fused_kernel,
        out_shape=jax.ShapeDtypeStruct((20, 256), jnp.float32),
    )(piece_ids.reshape(20, 1), orientations.reshape(20, 1),
      corner_slot_w, corner_piece_w, corner_orient_w,
      edge_slot_w, edge_piece_w, edge_orient_w,
      proj_w, proj_b.reshape(1, 256))
    return out.reshape(1, 20, 256)


# speedup vs baseline: 7.9464x; 1.5893x over previous
import jax
import jax.numpy as jnp
from jax.experimental import pallas as pl


def _fused_kernel(pid_ref, orient_ref, cslot_ref, cpiece_ref, corient_ref,
                  eslot_ref, epiece_ref, eorient_ref, proj_w_ref, proj_b_ref,
                  out_ref):
    pid = jnp.transpose(pid_ref[...])
    oid = jnp.transpose(orient_ref[...])

    slot_all = jnp.concatenate([cslot_ref[...], eslot_ref[...]], axis=0)
    piece_all = jnp.concatenate([cpiece_ref[...], epiece_ref[...]], axis=0)
    orient_all = jnp.concatenate([corient_ref[...], eorient_ref[...]], axis=0)

    row = jax.lax.broadcasted_iota(jnp.int32, (20, 1), 0)
    oid_adj = oid + jnp.where(row >= 8, 3, 0)

    k20 = jax.lax.broadcasted_iota(jnp.int32, (20, 20), 1)
    onehot_p = (pid == k20).astype(jnp.float32)
    emb_piece = jnp.dot(onehot_p, piece_all,
                        preferred_element_type=jnp.float32)

    k5 = jax.lax.broadcasted_iota(jnp.int32, (20, 5), 1)
    onehot_o = (oid_adj == k5).astype(jnp.float32)
    emb_orient = jnp.dot(onehot_o, orient_all,
                         preferred_element_type=jnp.float32)

    embedded = jnp.concatenate(
        [slot_all, emb_piece, emb_orient], axis=1)

    out_ref[...] = (jnp.dot(embedded, proj_w_ref[...],
                            preferred_element_type=jnp.float32)
                    + proj_b_ref[...].reshape(1, 256))


def kernel(piece_ids, orientations, corner_slot_w, corner_piece_w,
           corner_orient_w, edge_slot_w, edge_piece_w, edge_orient_w,
           proj_w, proj_b):
    out = pl.pallas_call(
        _fused_kernel,
        out_shape=jax.ShapeDtypeStruct((20, 256), jnp.float32),
    )(piece_ids, orientations,
      corner_slot_w, corner_piece_w, corner_orient_w,
      edge_slot_w, edge_piece_w, edge_orient_w,
      proj_w, proj_b)
    return out.reshape(1, 20, 256)


# speedup vs baseline: 8.2559x; 1.0389x over previous
import jax
import jax.numpy as jnp
from jax.experimental import pallas as pl
from jax.experimental.pallas import tpu as pltpu


def _fused_kernel(pid_ref, orient_ref, cslot_ref, cpiece_ref, corient_ref,
                  eslot_ref, epiece_ref, eorient_ref, proj_w_ref, proj_b_ref,
                  out_ref):
    row = jax.lax.broadcasted_iota(jnp.int32, (20, 1), 0)
    pid = jnp.zeros((20, 1), jnp.int32)
    oid = jnp.zeros((20, 1), jnp.int32)
    for r in range(20):
        pid = jnp.where(row == r, pid_ref[0, r], pid)
        oid = jnp.where(row == r, orient_ref[0, r], oid)

    slot_all = jnp.concatenate([cslot_ref[...], eslot_ref[...]], axis=0)
    piece_all = jnp.concatenate([cpiece_ref[...], epiece_ref[...]], axis=0)
    orient_all = jnp.concatenate([corient_ref[...], eorient_ref[...]], axis=0)

    oid_adj = oid + jnp.where(row >= 8, 3, 0)

    k20 = jax.lax.broadcasted_iota(jnp.int32, (20, 20), 1)
    onehot_p = (pid == k20).astype(jnp.float32)
    emb_piece = jnp.dot(onehot_p, piece_all,
                        preferred_element_type=jnp.float32)

    k5 = jax.lax.broadcasted_iota(jnp.int32, (20, 5), 1)
    onehot_o = (oid_adj == k5).astype(jnp.float32)
    emb_orient = jnp.dot(onehot_o, orient_all,
                         preferred_element_type=jnp.float32)

    embedded = jnp.concatenate(
        [slot_all, emb_piece, emb_orient], axis=1)

    out_ref[...] = (jnp.dot(embedded, proj_w_ref[...],
                            preferred_element_type=jnp.float32)
                    + proj_b_ref[...].reshape(1, 256))


def kernel(piece_ids, orientations, corner_slot_w, corner_piece_w,
           corner_orient_w, edge_slot_w, edge_piece_w, edge_orient_w,
           proj_w, proj_b):
    smem = pl.BlockSpec(memory_space=pltpu.SMEM)
    vmem = pl.BlockSpec(memory_space=pltpu.VMEM)
    out = pl.pallas_call(
        _fused_kernel,
        in_specs=[smem, smem] + [vmem] * 8,
        out_shape=jax.ShapeDtypeStruct((20, 256), jnp.float32),
    )(piece_ids, orientations,
      corner_slot_w, corner_piece_w, corner_orient_w,
      edge_slot_w, edge_piece_w, edge_orient_w,
      proj_w, proj_b)
    return out.reshape(1, 20, 256)
